# baseline (speedup 1.0000x reference)
import functools

import jax
import jax.numpy as jnp
from jax.experimental import pallas as pl
from jax.experimental.pallas import tpu as pltpu


def _fused_kernel(a_ref, x_ref, w_ref, bias_ref, out_ref, xw_ref, xw2_ref,
                  *, bm, n):
    i = pl.program_id(0)

    @pl.when(i == 0)
    def _compute_xw():
        xw = jnp.dot(x_ref[...], w_ref[...],
                     preferred_element_type=jnp.float32)
        xw_ref[...] = xw.astype(jnp.bfloat16)
        xw2_ref[...] = (xw * xw).astype(jnp.bfloat16)

    a = a_ref[...]
    rows = i * bm + jax.lax.broadcasted_iota(jnp.int32, (bm, n), 0)
    cols = jax.lax.broadcasted_iota(jnp.int32, (bm, n), 1)
    a = a + jnp.where(rows == cols, 1.0, 0.0).astype(a.dtype)
    a2 = a * a

    s = jnp.dot(a.astype(jnp.bfloat16), xw_ref[...],
                preferred_element_type=jnp.float32)
    t = jnp.dot(a2.astype(jnp.bfloat16), xw2_ref[...],
                preferred_element_type=jnp.float32)
    rs = jnp.sum(a, axis=1, keepdims=True)
    rss = jnp.sum(a2, axis=1, keepdims=True)

    denom = rs * rs - rss
    inv = 1.0 / denom
    inv = jnp.where(jnp.isinf(inv), 0.0, inv)
    out_ref[...] = inv * (s * s - t) + bias_ref[...]


def _pick_block(n, cap):
    best = 1
    for d in range(1, n + 1):
        if n % d == 0 and d <= cap and d % 8 == 0:
            best = d
    return best if n % 8 == 0 else n


def kernel(x, edge_index, edge_weight, weight, bias):
    del edge_weight
    n, d_in = x.shape
    d_out = weight.shape[1]

    bm = _pick_block(n, 400)
    grid = (n // bm,)

    out = pl.pallas_call(
        functools.partial(_fused_kernel, bm=bm, n=n),
        grid=grid,
        in_specs=[
            pl.BlockSpec((bm, n), lambda i: (i, 0)),
            pl.BlockSpec((n, d_in), lambda i: (0, 0)),
            pl.BlockSpec((d_in, d_out), lambda i: (0, 0)),
            pl.BlockSpec((1, d_out), lambda i: (0, 0)),
        ],
        out_specs=pl.BlockSpec((bm, d_out), lambda i: (i, 0)),
        out_shape=jax.ShapeDtypeStruct((n, d_out), jnp.float32),
        scratch_shapes=[
            pltpu.VMEM((n, d_out), jnp.bfloat16),
            pltpu.VMEM((n, d_out), jnp.bfloat16),
        ],
        compiler_params=pltpu.CompilerParams(
            dimension_semantics=("arbitrary",),

        ),
    )(edge_index, x, weight, bias.reshape(1, d_out))

    return out


# speedup vs baseline: 1.0193x; 1.0193x over previous
import functools

import jax
import jax.numpy as jnp
from jax.experimental import pallas as pl
from jax.experimental.pallas import tpu as pltpu


def _fused_kernel(a_ref, x_ref, w_ref, bias_ref, out_ref, xwa_ref, xw2a_ref,
                  *, bm, n, d_out):
    i = pl.program_id(0)

    @pl.when(i == 0)
    def _compute_xw():
        xw = jnp.dot(x_ref[...], w_ref[...],
                     preferred_element_type=jnp.float32)
        ones_col = jnp.where(
            jax.lax.broadcasted_iota(jnp.int32, (n, d_out), 1) == 0,
            1.0, 0.0)
        xwa_ref[...] = jnp.concatenate(
            [xw, ones_col], axis=1).astype(jnp.bfloat16)
        xw2a_ref[...] = jnp.concatenate(
            [xw * xw, ones_col], axis=1).astype(jnp.bfloat16)

    a = a_ref[...]
    rows = i * bm + jax.lax.broadcasted_iota(jnp.int32, (bm, n), 0)
    cols = jax.lax.broadcasted_iota(jnp.int32, (bm, n), 1)
    ab = (a + jnp.where(rows == cols, 1.0, 0.0)).astype(jnp.bfloat16)
    a2b = ab * ab

    saug = jnp.dot(ab, xwa_ref[...], preferred_element_type=jnp.float32)
    taug = jnp.dot(a2b, xw2a_ref[...], preferred_element_type=jnp.float32)
    s = saug[:, :d_out]
    t = taug[:, :d_out]
    rs = saug[:, d_out:d_out + 1]
    rss = taug[:, d_out:d_out + 1]

    denom = rs * rs - rss
    inv = 1.0 / denom
    inv = jnp.where(jnp.isinf(inv), 0.0, inv)
    out_ref[...] = inv * (s * s - t) + bias_ref[...]


def _pick_block(n, cap):
    best = 1
    for d in range(1, n + 1):
        if n % d == 0 and d <= cap and d % 8 == 0:
            best = d
    return best if n % 8 == 0 else n


def kernel(x, edge_index, edge_weight, weight, bias):
    del edge_weight
    n, d_in = x.shape
    d_out = weight.shape[1]

    bm = _pick_block(n, 400)
    grid = (n // bm,)

    out = pl.pallas_call(
        functools.partial(_fused_kernel, bm=bm, n=n, d_out=d_out),
        grid=grid,
        in_specs=[
            pl.BlockSpec((bm, n), lambda i: (i, 0)),
            pl.BlockSpec((n, d_in), lambda i: (0, 0)),
            pl.BlockSpec((d_in, d_out), lambda i: (0, 0)),
            pl.BlockSpec((1, d_out), lambda i: (0, 0)),
        ],
        out_specs=pl.BlockSpec((bm, d_out), lambda i: (i, 0)),
        out_shape=jax.ShapeDtypeStruct((n, d_out), jnp.float32),
        scratch_shapes=[
            pltpu.VMEM((n, 2 * d_out), jnp.bfloat16),
            pltpu.VMEM((n, 2 * d_out), jnp.bfloat16),
        ],
        compiler_params=pltpu.CompilerParams(
            dimension_semantics=("arbitrary",),
        ),
    )(edge_index, x, weight, bias.reshape(1, d_out))

    return out
